# baseline (speedup 1.0000x reference)
import jax
import jax.numpy as jnp
from jax import lax
from jax.experimental import pallas as pl
from jax.experimental.pallas import tpu as pltpu
from jax.experimental.pallas import tpu_sc as plsc

N_PAD = 10240
STRIPE = N_PAD // 16
CHUNK = 80


def _sc_mesh():
    return plsc.VectorSubcoreMesh(core_axis_name="c", subcore_axis_name="s")


_DEG_K = 5


def _deg_kernel(E):
    per_w = E // 32
    DEG_E = _DEG_K * CHUNK
    n_stages = per_w // DEG_E
    u_iters = (n_stages - 1) // 4
    assert n_stages == 4 * u_iters + 1

    def body(dst_hbm, ones_hbm, zeros_hbm, out_hbm, acc, ones_v, *rest):
        didx = rest[0:4]
        isem = rest[4:8]
        ssem = rest[8]
        c = lax.axis_index("c")
        s = lax.axis_index("s")
        wid = s * 2 + c
        wbase = wid * per_w

        def issue_idx(m, stage):
            pltpu.async_copy(dst_hbm.at[pl.ds(wbase + stage * DEG_E, DEG_E)],
                             didx[m], isem[m])

        def drain_idx(m):
            pltpu.make_async_copy(dst_hbm.at[pl.ds(0, DEG_E)],
                                  didx[m], isem[m]).wait()

        def drain_scat():
            pltpu.make_async_copy(dst_hbm.at[pl.ds(0, DEG_E)],
                                  didx[0], ssem).wait()

        def stage(m, stage_idx, drain_guard, issue_guard):
            drain_idx(m)
            for k in range(_DEG_K):
                pltpu.async_copy(ones_v,
                                 acc.at[didx[m].at[pl.ds(k * CHUNK, CHUNK)]],
                                 ssem, add=True)
            if drain_guard is None:
                drain_scat()
            else:
                pl.when(drain_guard)(drain_scat)
            issue = lambda: issue_idx((m + 2) % 4, stage_idx + 2)
            if issue_guard is None:
                issue()
            else:
                pl.when(issue_guard)(issue)

        pltpu.sync_copy(zeros_hbm.at[pl.ds(s * STRIPE, STRIPE)],
                        acc.at[pl.ds(s * STRIPE, STRIPE)])
        pltpu.sync_copy(ones_hbm.at[pl.ds(0, CHUNK)], ones_v)
        issue_idx(0, 0)
        issue_idx(1, 1)
        plsc.subcore_barrier()

        def step(j, carry):
            sbase = 4 * j
            stage(0, sbase + 0, j > 0, None)
            stage(1, sbase + 1, j > 0, None)
            stage(2, sbase + 2, None, None)
            stage(3, sbase + 3, None, j < u_iters - 1)
            return carry

        lax.fori_loop(0, u_iters, step, 0)
        drain_idx(0)
        for k in range(_DEG_K):
            pltpu.async_copy(ones_v,
                             acc.at[didx[0].at[pl.ds(k * CHUNK, CHUNK)]],
                             ssem, add=True)
        drain_scat()
        drain_scat()
        drain_scat()
        plsc.subcore_barrier()
        pltpu.sync_copy(acc.at[pl.ds(s * STRIPE, STRIPE)],
                        out_hbm.at[pl.ds(c * N_PAD + s * STRIPE, STRIPE)])

    return pl.kernel(
        body,
        out_type=jax.ShapeDtypeStruct((2 * N_PAD,), jnp.float32),
        mesh=_sc_mesh(),
        scratch_types=[
            pltpu.VMEM_SHARED((N_PAD,), jnp.float32),
            pltpu.VMEM((CHUNK,), jnp.float32),
        ] + [pltpu.VMEM((DEG_E,), jnp.int32)] * 4
          + [pltpu.SemaphoreType.DMA] * 5,
    )


_SEG_K = 2


def _segsum_kernel(E, D):
    per_w = E // 32
    per_w_chunks = per_w // CHUNK
    n_iters = per_w_chunks // (2 * _SEG_K)
    n_stages = n_iters * 2
    tail = per_w_chunks - n_stages * _SEG_K
    SEG_E = _SEG_K * CHUNK

    assert n_iters % 2 == 1 and n_iters >= 3
    u_iters = (n_iters - 1) // 2

    def body(g_hbm, src_hbm, dst_hbm, zeros_hbm, out_hbm, acc, *rest):
        sidx = rest[0:4]
        didx = rest[4:8]
        rows = rest[8:10]
        sems = rest[10:]
        isem = sems[0:4]
        ssem = sems[4:6]
        gsem = (sems[6:6 + _SEG_K], sems[6 + _SEG_K:6 + 2 * _SEG_K])
        c = lax.axis_index("c")
        s = lax.axis_index("s")
        wid = s * 2 + c
        wbase = wid * per_w

        def issue_idx(m, stage):
            local = jnp.minimum(stage * SEG_E, per_w - SEG_E)
            base = wbase + local
            pltpu.async_copy(src_hbm.at[pl.ds(base, SEG_E)], sidx[m], isem[m])
            pltpu.async_copy(dst_hbm.at[pl.ds(base, SEG_E)], didx[m], isem[m])

        def drain_idx(m):
            pltpu.make_async_copy(src_hbm.at[pl.ds(0, SEG_E)],
                                  sidx[m], isem[m]).wait()
            pltpu.make_async_copy(dst_hbm.at[pl.ds(0, SEG_E)],
                                  didx[m], isem[m]).wait()

        def drain_scat(p):
            for k in range(_SEG_K):
                pltpu.make_async_copy(g_hbm.at[pl.ds(0, CHUNK), :],
                                      rows[p].at[k], ssem[p]).wait()

        def stage(p, m, stage_idx, guard, do_issue=True):
            drain_idx(m)
            gd = [pltpu.async_copy(
                g_hbm.at[sidx[m].at[pl.ds(k * CHUNK, CHUNK)]],
                rows[p].at[k], gsem[p][k]) for k in range(_SEG_K)]
            for k in range(_SEG_K):
                gd[k].wait()
                pltpu.async_copy(rows[p].at[k],
                                 acc.at[didx[m].at[pl.ds(k * CHUNK, CHUNK)]],
                                 ssem[p], add=True)

            def after():
                drain_scat(1 - p)
                if do_issue:
                    issue_idx((m + 3) % 4, stage_idx + 3)

            if guard is None:
                after()
            else:
                pl.when(guard)(after)

        pltpu.sync_copy(zeros_hbm.at[pl.ds(s * STRIPE, STRIPE), :],
                        acc.at[pl.ds(s * STRIPE, STRIPE), :])
        for m in range(4):
            issue_idx(m, m)
        plsc.subcore_barrier()

        def step(j, carry):
            sbase = 4 * j
            stage(0, 0, sbase + 0, j > 0)
            stage(1, 1, sbase + 1, None)
            stage(0, 2, sbase + 2, None)
            stage(1, 3, sbase + 3, None)
            return carry

        lax.fori_loop(0, u_iters, step, 0)
        sbase = 4 * u_iters
        stage(0, 0, sbase + 0, None, do_issue=False)
        stage(1, 1, sbase + 1, None, do_issue=False)
        drain_scat(1)
        drain_idx(2)
        for t in range(tail):
            base = wbase + (n_stages * _SEG_K + t) * CHUNK
            pltpu.sync_copy(src_hbm.at[pl.ds(base, CHUNK)],
                            sidx[0].at[pl.ds(0, CHUNK)])
            pltpu.sync_copy(dst_hbm.at[pl.ds(base, CHUNK)],
                            didx[0].at[pl.ds(0, CHUNK)])
            pltpu.async_copy(g_hbm.at[sidx[0].at[pl.ds(0, CHUNK)]],
                             rows[0].at[0], gsem[0][0]).wait()
            pltpu.async_copy(rows[0].at[0],
                             acc.at[didx[0].at[pl.ds(0, CHUNK)]],
                             ssem[0], add=True).wait()
        plsc.subcore_barrier()
        pltpu.sync_copy(acc.at[pl.ds(s * STRIPE, STRIPE), :],
                        out_hbm.at[pl.ds(c * N_PAD + s * STRIPE, STRIPE), :])

    return pl.kernel(
        body,
        out_type=jax.ShapeDtypeStruct((2 * N_PAD, D), jnp.float32),
        mesh=_sc_mesh(),
        scratch_types=(
            [pltpu.VMEM_SHARED((N_PAD, D), jnp.float32)]
            + [pltpu.VMEM((SEG_E,), jnp.int32)] * 8
            + [pltpu.VMEM((_SEG_K, CHUNK, D), jnp.float32)] * 2
            + [pltpu.SemaphoreType.DMA] * (6 + 2 * _SEG_K)
        ),
    )


_BLK = 1000


def _k2_body(d0_ref, d1_ref, x_ref, w_ref, g_ref, dinv_ref):
    dinv = lax.rsqrt(d0_ref[...] + d1_ref[...] + 1.0)
    h = jnp.dot(x_ref[...], w_ref[...], preferred_element_type=jnp.float32)
    g_ref[...] = h * dinv
    dinv_ref[...] = dinv


def _k4_body(t0_ref, t1_ref, g1_ref, dinv_ref, b1_ref, w2_ref, g2_ref):
    dinv = dinv_ref[...]
    h1 = jnp.maximum(dinv * (t0_ref[...] + t1_ref[...] + g1_ref[...])
                     + b1_ref[...], 0.0)
    g2_ref[...] = jnp.dot(h1, w2_ref[...],
                          preferred_element_type=jnp.float32) * dinv


def _k6_body(t0_ref, t1_ref, g2_ref, dinv_ref, b2_ref, wl1_ref, bl1_ref,
             wl2_ref, bl2_ref, out_ref, h_ref):
    h = dinv_ref[...] * (t0_ref[...] + t1_ref[...] + g2_ref[...]) + b2_ref[...]
    h_ref[...] = h
    a = jnp.maximum(jnp.dot(h, wl1_ref[...],
                            preferred_element_type=jnp.float32)
                    + bl1_ref[...], 0.0)
    logits = jnp.dot(a, wl2_ref[...],
                     preferred_element_type=jnp.float32) + bl2_ref[...]
    m = jnp.max(logits, axis=-1, keepdims=True)
    lse = jnp.log(jnp.sum(jnp.exp(logits - m), axis=-1, keepdims=True)) + m
    out_ref[...] = logits - lse


def _row_spec(w):
    return pl.BlockSpec((_BLK, w), lambda i: (i, 0))


def _full_spec(h, w):
    return pl.BlockSpec((h, w), lambda i: (0, 0))


def kernel(x, edge_index, epoch, W1, b1, W2, b2, Wl1, bl1, Wl2, bl2):
    N, D = x.shape
    E = edge_index.shape[1]
    C = Wl2.shape[1]
    src = edge_index[0]
    dst = edge_index[1]

    ones_e = jnp.ones((CHUNK,), jnp.float32)
    zeros_v = jnp.zeros((N_PAD,), jnp.float32)
    zeros_r = jnp.zeros((N_PAD, D), jnp.float32)

    grid = N // _BLK

    degp = _deg_kernel(E)(dst, ones_e, zeros_v)
    d0 = degp[:N].reshape(N, 1)
    d1 = degp[N_PAD:N_PAD + N].reshape(N, 1)

    g1, dinv = pl.pallas_call(
        _k2_body,
        grid=(grid,),
        in_specs=[_row_spec(1), _row_spec(1), _row_spec(D), _full_spec(D, D)],
        out_specs=[_row_spec(D), _row_spec(1)],
        out_shape=[jax.ShapeDtypeStruct((N, D), jnp.float32),
                   jax.ShapeDtypeStruct((N, 1), jnp.float32)],
    )(d0, d1, x, W1)

    t1p = _segsum_kernel(E, D)(g1, src, dst, zeros_r)
    t10 = t1p[:N]
    t11 = t1p[N_PAD:N_PAD + N]

    g2 = pl.pallas_call(
        _k4_body,
        grid=(grid,),
        in_specs=[_row_spec(D), _row_spec(D), _row_spec(D), _row_spec(1),
                  _full_spec(1, D), _full_spec(D, D)],
        out_specs=_row_spec(D),
        out_shape=jax.ShapeDtypeStruct((N, D), jnp.float32),
    )(t10, t11, g1, dinv, b1.reshape(1, D), W2)

    t2p = _segsum_kernel(E, D)(g2, src, dst, zeros_r)
    t20 = t2p[:N]
    t21 = t2p[N_PAD:N_PAD + N]

    out, h = pl.pallas_call(
        _k6_body,
        grid=(grid,),
        in_specs=[_row_spec(D), _row_spec(D), _row_spec(D), _row_spec(1),
                  _full_spec(1, D), _full_spec(D, D), _full_spec(1, D),
                  _full_spec(D, C), _full_spec(1, C)],
        out_specs=[_row_spec(C), _row_spec(D)],
        out_shape=[jax.ShapeDtypeStruct((N, C), jnp.float32),
                   jax.ShapeDtypeStruct((N, D), jnp.float32)],
    )(t20, t21, g2, dinv, b2.reshape(1, D), Wl1, bl1.reshape(1, D),
      Wl2, bl2.reshape(1, C))

    return (out, h, out)


# speedup vs baseline: 1.0180x; 1.0180x over previous
import jax
import jax.numpy as jnp
from jax import lax
from jax.experimental import pallas as pl
from jax.experimental.pallas import tpu as pltpu
from jax.experimental.pallas import tpu_sc as plsc

N_PAD = 10240
STRIPE = N_PAD // 16
CHUNK = 80


def _sc_mesh():
    return plsc.VectorSubcoreMesh(core_axis_name="c", subcore_axis_name="s")


_DEG_K = 5


def _deg_kernel(E):
    per_w = E // 32
    DEG_E = _DEG_K * CHUNK
    n_stages = per_w // DEG_E
    u_iters = (n_stages - 1) // 4
    assert n_stages == 4 * u_iters + 1

    def body(dst_hbm, ones_hbm, zeros_hbm, out_hbm, acc, ones_v, *rest):
        didx = rest[0:4]
        isem = rest[4:8]
        ssem = rest[8]
        c = lax.axis_index("c")
        s = lax.axis_index("s")
        wid = s * 2 + c
        wbase = wid * per_w

        def issue_idx(m, stage):
            pltpu.async_copy(dst_hbm.at[pl.ds(wbase + stage * DEG_E, DEG_E)],
                             didx[m], isem[m])

        def drain_idx(m):
            pltpu.make_async_copy(dst_hbm.at[pl.ds(0, DEG_E)],
                                  didx[m], isem[m]).wait()

        def drain_scat():
            pltpu.make_async_copy(dst_hbm.at[pl.ds(0, DEG_E)],
                                  didx[0], ssem).wait()

        def stage(m, stage_idx, drain_guard, issue_guard):
            drain_idx(m)
            for k in range(_DEG_K):
                pltpu.async_copy(ones_v,
                                 acc.at[didx[m].at[pl.ds(k * CHUNK, CHUNK)]],
                                 ssem, add=True)
            if drain_guard is None:
                drain_scat()
            else:
                pl.when(drain_guard)(drain_scat)
            issue = lambda: issue_idx((m + 2) % 4, stage_idx + 2)
            if issue_guard is None:
                issue()
            else:
                pl.when(issue_guard)(issue)

        pltpu.sync_copy(zeros_hbm.at[pl.ds(s * STRIPE, STRIPE)],
                        acc.at[pl.ds(s * STRIPE, STRIPE)])
        pltpu.sync_copy(ones_hbm.at[pl.ds(0, CHUNK)], ones_v)
        issue_idx(0, 0)
        issue_idx(1, 1)
        plsc.subcore_barrier()

        def step(j, carry):
            sbase = 4 * j
            stage(0, sbase + 0, j > 0, None)
            stage(1, sbase + 1, j > 0, None)
            stage(2, sbase + 2, None, None)
            stage(3, sbase + 3, None, j < u_iters - 1)
            return carry

        lax.fori_loop(0, u_iters, step, 0)
        drain_idx(0)
        for k in range(_DEG_K):
            pltpu.async_copy(ones_v,
                             acc.at[didx[0].at[pl.ds(k * CHUNK, CHUNK)]],
                             ssem, add=True)
        drain_scat()
        drain_scat()
        drain_scat()
        plsc.subcore_barrier()
        pltpu.sync_copy(acc.at[pl.ds(s * STRIPE, STRIPE)],
                        out_hbm.at[pl.ds(c * N_PAD + s * STRIPE, STRIPE)])

    return pl.kernel(
        body,
        out_type=jax.ShapeDtypeStruct((2 * N_PAD,), jnp.float32),
        mesh=_sc_mesh(),
        scratch_types=[
            pltpu.VMEM_SHARED((N_PAD,), jnp.float32),
            pltpu.VMEM((CHUNK,), jnp.float32),
        ] + [pltpu.VMEM((DEG_E,), jnp.int32)] * 4
          + [pltpu.SemaphoreType.DMA] * 5,
    )


_SEG_K = 2


def _segsum_kernel(E, D):
    per_w = E // 32
    per_w_chunks = per_w // CHUNK
    n_iters = per_w_chunks // (2 * _SEG_K)
    n_stages = n_iters * 2
    tail = per_w_chunks - n_stages * _SEG_K
    SEG_E = _SEG_K * CHUNK

    assert n_iters % 2 == 1 and n_iters >= 3
    u_iters = (n_iters - 1) // 2

    def body(g_hbm, src_hbm, dst_hbm, zeros_hbm, out_hbm, acc, *rest):
        sidx = rest[0:4]
        didx = rest[4:8]
        rows = rest[8:10]
        sems = rest[10:]
        isem = sems[0:4]
        ssem = sems[4:6]
        gsem = (sems[6:6 + _SEG_K], sems[6 + _SEG_K:6 + 2 * _SEG_K])
        c = lax.axis_index("c")
        s = lax.axis_index("s")
        wid = s * 2 + c
        wbase = wid * per_w

        def issue_idx(m, stage):
            local = jnp.minimum(stage * SEG_E, per_w - SEG_E)
            base = wbase + local
            pltpu.async_copy(src_hbm.at[pl.ds(base, SEG_E)], sidx[m], isem[m])
            pltpu.async_copy(dst_hbm.at[pl.ds(base, SEG_E)], didx[m], isem[m])

        def drain_idx(m):
            pltpu.make_async_copy(src_hbm.at[pl.ds(0, SEG_E)],
                                  sidx[m], isem[m]).wait()
            pltpu.make_async_copy(dst_hbm.at[pl.ds(0, SEG_E)],
                                  didx[m], isem[m]).wait()

        def drain_scat(p):
            for k in range(_SEG_K):
                pltpu.make_async_copy(g_hbm.at[pl.ds(0, CHUNK), :],
                                      rows[p].at[k], ssem[p]).wait()

        def stage(p, m, stage_idx, guard, do_issue=True):
            drain_idx(m)
            gd = [pltpu.async_copy(
                g_hbm.at[sidx[m].at[pl.ds(k * CHUNK, CHUNK)]],
                rows[p].at[k], gsem[p][k]) for k in range(_SEG_K)]
            for k in range(_SEG_K):
                gd[k].wait()
                pltpu.async_copy(rows[p].at[k],
                                 acc.at[didx[m].at[pl.ds(k * CHUNK, CHUNK)]],
                                 ssem[p], add=True)

            def after():
                drain_scat(1 - p)
                if do_issue:
                    issue_idx((m + 3) % 4, stage_idx + 3)

            if guard is None:
                after()
            else:
                pl.when(guard)(after)

        pltpu.sync_copy(zeros_hbm.at[pl.ds(s * STRIPE, STRIPE), :],
                        acc.at[pl.ds(s * STRIPE, STRIPE), :])
        for m in range(4):
            issue_idx(m, m)
        plsc.subcore_barrier()

        def step(j, carry):
            sbase = 4 * j
            stage(0, 0, sbase + 0, j > 0)
            stage(1, 1, sbase + 1, None)
            stage(0, 2, sbase + 2, None)
            stage(1, 3, sbase + 3, None)
            return carry

        lax.fori_loop(0, u_iters, step, 0)
        sbase = 4 * u_iters
        stage(0, 0, sbase + 0, None, do_issue=False)
        stage(1, 1, sbase + 1, None, do_issue=False)
        drain_scat(1)
        drain_idx(2)
        for t in range(tail):
            base = wbase + (n_stages * _SEG_K + t) * CHUNK
            pltpu.sync_copy(src_hbm.at[pl.ds(base, CHUNK)],
                            sidx[0].at[pl.ds(0, CHUNK)])
            pltpu.sync_copy(dst_hbm.at[pl.ds(base, CHUNK)],
                            didx[0].at[pl.ds(0, CHUNK)])
            pltpu.async_copy(g_hbm.at[sidx[0].at[pl.ds(0, CHUNK)]],
                             rows[0].at[0], gsem[0][0]).wait()
            pltpu.async_copy(rows[0].at[0],
                             acc.at[didx[0].at[pl.ds(0, CHUNK)]],
                             ssem[0], add=True).wait()
        plsc.subcore_barrier()
        pltpu.sync_copy(acc.at[pl.ds(s * STRIPE, STRIPE), :],
                        out_hbm.at[pl.ds(c * N_PAD + s * STRIPE, STRIPE), :])

    return pl.kernel(
        body,
        out_type=jax.ShapeDtypeStruct((2 * N_PAD, D), jnp.float32),
        mesh=_sc_mesh(),
        scratch_types=(
            [pltpu.VMEM_SHARED((N_PAD, D), jnp.float32)]
            + [pltpu.VMEM((SEG_E,), jnp.int32)] * 8
            + [pltpu.VMEM((_SEG_K, CHUNK, D), jnp.float32)] * 2
            + [pltpu.SemaphoreType.DMA] * (6 + 2 * _SEG_K)
        ),
    )


_BLK = 2000


def _k2_body(d0_ref, d1_ref, x_ref, w_ref, g_ref, dinv_ref):
    dinv = lax.rsqrt(d0_ref[...] + d1_ref[...] + 1.0)
    h = jnp.dot(x_ref[...], w_ref[...], preferred_element_type=jnp.float32)
    g_ref[...] = h * dinv
    dinv_ref[...] = dinv


def _k4_body(t0_ref, t1_ref, g1_ref, dinv_ref, b1_ref, w2_ref, g2_ref):
    dinv = dinv_ref[...]
    h1 = jnp.maximum(dinv * (t0_ref[...] + t1_ref[...] + g1_ref[...])
                     + b1_ref[...], 0.0)
    g2_ref[...] = jnp.dot(h1, w2_ref[...],
                          preferred_element_type=jnp.float32) * dinv


def _k6_body(t0_ref, t1_ref, g2_ref, dinv_ref, b2_ref, wl1_ref, bl1_ref,
             wl2_ref, bl2_ref, out_ref, h_ref):
    h = dinv_ref[...] * (t0_ref[...] + t1_ref[...] + g2_ref[...]) + b2_ref[...]
    h_ref[...] = h
    a = jnp.maximum(jnp.dot(h, wl1_ref[...],
                            preferred_element_type=jnp.float32)
                    + bl1_ref[...], 0.0)
    logits = jnp.dot(a, wl2_ref[...],
                     preferred_element_type=jnp.float32) + bl2_ref[...]
    m = jnp.max(logits, axis=-1, keepdims=True)
    lse = jnp.log(jnp.sum(jnp.exp(logits - m), axis=-1, keepdims=True)) + m
    out_ref[...] = logits - lse


def _row_spec(w):
    return pl.BlockSpec((_BLK, w), lambda i: (i, 0))


def _full_spec(h, w):
    return pl.BlockSpec((h, w), lambda i: (0, 0))


def kernel(x, edge_index, epoch, W1, b1, W2, b2, Wl1, bl1, Wl2, bl2):
    N, D = x.shape
    E = edge_index.shape[1]
    C = Wl2.shape[1]
    src = edge_index[0]
    dst = edge_index[1]

    ones_e = jnp.ones((CHUNK,), jnp.float32)
    zeros_v = jnp.zeros((N_PAD,), jnp.float32)
    zeros_r = jnp.zeros((N_PAD, D), jnp.float32)

    grid = N // _BLK

    degp = _deg_kernel(E)(dst, ones_e, zeros_v)
    d0 = degp[:N].reshape(N, 1)
    d1 = degp[N_PAD:N_PAD + N].reshape(N, 1)

    g1, dinv = pl.pallas_call(
        _k2_body,
        grid=(grid,),
        in_specs=[_row_spec(1), _row_spec(1), _row_spec(D), _full_spec(D, D)],
        out_specs=[_row_spec(D), _row_spec(1)],
        out_shape=[jax.ShapeDtypeStruct((N, D), jnp.float32),
                   jax.ShapeDtypeStruct((N, 1), jnp.float32)],
    )(d0, d1, x, W1)

    t1p = _segsum_kernel(E, D)(g1, src, dst, zeros_r)
    t10 = t1p[:N]
    t11 = t1p[N_PAD:N_PAD + N]

    g2 = pl.pallas_call(
        _k4_body,
        grid=(grid,),
        in_specs=[_row_spec(D), _row_spec(D), _row_spec(D), _row_spec(1),
                  _full_spec(1, D), _full_spec(D, D)],
        out_specs=_row_spec(D),
        out_shape=jax.ShapeDtypeStruct((N, D), jnp.float32),
    )(t10, t11, g1, dinv, b1.reshape(1, D), W2)

    t2p = _segsum_kernel(E, D)(g2, src, dst, zeros_r)
    t20 = t2p[:N]
    t21 = t2p[N_PAD:N_PAD + N]

    out, h = pl.pallas_call(
        _k6_body,
        grid=(grid,),
        in_specs=[_row_spec(D), _row_spec(D), _row_spec(D), _row_spec(1),
                  _full_spec(1, D), _full_spec(D, D), _full_spec(1, D),
                  _full_spec(D, C), _full_spec(1, C)],
        out_specs=[_row_spec(C), _row_spec(D)],
        out_shape=[jax.ShapeDtypeStruct((N, C), jnp.float32),
                   jax.ShapeDtypeStruct((N, D), jnp.float32)],
    )(t20, t21, g2, dinv, b2.reshape(1, D), Wl1, bl1.reshape(1, D),
      Wl2, bl2.reshape(1, C))

    return (out, h, out)
